# baseline (speedup 1.0000x reference)
import functools

import jax
import jax.numpy as jnp
from jax import lax
from jax.experimental import pallas as pl
from jax.experimental.pallas import tpu as pltpu
from jax.experimental.pallas import tpu_sc as plsc

V, D, H = 100000, 128, 128
B, L = 1024, 200
TOK = B * L
NC, NS = 2, 16
NW = NC * NS
PER_W = TOK // NW
CH = 128
NCH = PER_W // CH
NB = 5


def _make_gather():
  mesh = plsc.VectorSubcoreMesh(core_axis_name="c", subcore_axis_name="s")

  @functools.partial(
      pl.kernel,
      mesh=mesh,
      out_type=jax.ShapeDtypeStruct((TOK, D), jnp.float32),
      scratch_types=[
          pltpu.VMEM((NCH, CH), jnp.int32),
      ] + [pltpu.VMEM((CH, D), jnp.float32)] * NB
        + [pltpu.SemaphoreType.DMA] * (2 * NB),
  )
  def gather_kernel(table_hbm, idx_hbm, out_hbm, idx_v, *bufs_and_sems):
    bufs = bufs_and_sems[:NB]
    gsems = bufs_and_sems[NB:2 * NB]
    wsems = bufs_and_sems[2 * NB:]
    wid = lax.axis_index("s") * NC + lax.axis_index("c")
    base = wid * PER_W
    pltpu.sync_copy(idx_hbm.at[wid], idx_v)

    for j in range(NB):
      pltpu.async_copy(table_hbm.at[idx_v.at[j]], bufs[j], gsems[j])

    def body(i, carry):
      c0 = NB * i
      for j in range(NB):
        pltpu.make_async_copy(
            table_hbm.at[idx_v.at[c0 + j]], bufs[j], gsems[j]).wait()
        pltpu.async_copy(
            bufs[j], out_hbm.at[pl.ds(base + (c0 + j) * CH, CH)], wsems[j])
      for j in range(NB):
        pltpu.make_async_copy(
            bufs[j], out_hbm.at[pl.ds(base + (c0 + j) * CH, CH)],
            wsems[j]).wait()

        @pl.when(i < NCH // NB - 1)
        def _():
          pltpu.async_copy(
              table_hbm.at[idx_v.at[c0 + NB + j]], bufs[j], gsems[j])

      return carry

    lax.fori_loop(0, NCH // NB, body, 0)

  return gather_kernel


_gather = _make_gather()

TBLK = 10000


def _highway_body(e_ref, wp_ref, wt0_ref, bt0_ref, wg0_ref, bg0_ref,
                  wt1_ref, bt1_ref, wg1_ref, bg1_ref, out_ref):
  dn = (((1,), (1,)), ((), ()))

  def mm(a, w_ref):
    return lax.dot_general(a.astype(jnp.bfloat16),
                           w_ref[...].astype(jnp.bfloat16), dn,
                           preferred_element_type=jnp.float32)

  h = mm(e_ref[...], wp_ref)
  for wt_ref, bt_ref, wg_ref, bg_ref in (
      (wt0_ref, bt0_ref, wg0_ref, bg0_ref),
      (wt1_ref, bt1_ref, wg1_ref, bg1_ref)):
    zg = mm(h, wg_ref) + bg_ref[...]
    zt = mm(h, wt_ref) + bt_ref[...]
    g = 1.0 / (1.0 + jnp.exp(-zg))
    t = jnp.maximum(zt, 0.0)
    h = g * t + (1.0 - g) * h
  out_ref[...] = h


def _make_highway():
  w_spec = pl.BlockSpec((H, H), lambda i: (0, 0))
  b_spec = pl.BlockSpec((1, H), lambda i: (0, 0))
  return pl.pallas_call(
      _highway_body,
      grid=(V // TBLK,),
      in_specs=[
          pl.BlockSpec((TBLK, D), lambda i: (i, 0)),
          w_spec, w_spec, b_spec, w_spec, b_spec,
          w_spec, b_spec, w_spec, b_spec,
      ],
      out_specs=pl.BlockSpec((TBLK, H), lambda i: (i, 0)),
      out_shape=jax.ShapeDtypeStruct((V, H), jnp.float32),
  )


_highway = _make_highway()


def kernel(x, word_vectors, W_proj, Wt0, bt0, Wg0, bg0, Wt1, bt1, Wg1, bg1):
  idx = x.reshape(NW, NCH, CH).astype(jnp.int32)
  ftable = _highway(word_vectors, W_proj,
                    Wt0, bt0.reshape(1, H), Wg0, bg0.reshape(1, H),
                    Wt1, bt1.reshape(1, H), Wg1, bg1.reshape(1, H))
  out = _gather(ftable, idx)
  return out.reshape(B, L, H)


# speedup vs baseline: 1.1300x; 1.1300x over previous
import functools

import jax
import jax.numpy as jnp
from jax import lax
from jax.experimental import pallas as pl
from jax.experimental.pallas import tpu as pltpu
from jax.experimental.pallas import tpu_sc as plsc

V, D, H = 100000, 128, 128
B, L = 1024, 200
TOK = B * L
NC, NS = 2, 16
NW = NC * NS
PER_W = TOK // NW
CH = 64
NCH = PER_W // CH
NB = 10


def _make_gather():
  mesh = plsc.VectorSubcoreMesh(core_axis_name="c", subcore_axis_name="s")

  @functools.partial(
      pl.kernel,
      mesh=mesh,
      out_type=jax.ShapeDtypeStruct((TOK, D), jnp.float32),
      scratch_types=[
          pltpu.VMEM((NCH, CH), jnp.int32),
      ] + [pltpu.VMEM((CH, D), jnp.float32)] * NB
        + [pltpu.SemaphoreType.DMA] * (2 * NB),
  )
  def gather_kernel(table_hbm, idx_hbm, out_hbm, idx_v, *bufs_and_sems):
    bufs = bufs_and_sems[:NB]
    gsems = bufs_and_sems[NB:2 * NB]
    wsems = bufs_and_sems[2 * NB:]
    wid = lax.axis_index("s") * NC + lax.axis_index("c")
    base = wid * PER_W
    pltpu.sync_copy(idx_hbm.at[wid], idx_v)

    for j in range(NB):
      pltpu.async_copy(table_hbm.at[idx_v.at[j]], bufs[j], gsems[j])

    def body(i, carry):
      c0 = NB * i
      for j in range(NB):
        pltpu.make_async_copy(
            table_hbm.at[idx_v.at[c0 + j]], bufs[j], gsems[j]).wait()
        pltpu.async_copy(
            bufs[j], out_hbm.at[pl.ds(base + (c0 + j) * CH, CH)], wsems[j])
      for j in range(NB):
        pltpu.make_async_copy(
            bufs[j], out_hbm.at[pl.ds(base + (c0 + j) * CH, CH)],
            wsems[j]).wait()

        @pl.when(i < NCH // NB - 1)
        def _():
          pltpu.async_copy(
              table_hbm.at[idx_v.at[c0 + NB + j]], bufs[j], gsems[j])

      return carry

    lax.fori_loop(0, NCH // NB, body, 0)

  return gather_kernel


_gather = _make_gather()

TBLK = 10000


def _highway_body(e_ref, wp_ref, wt0_ref, bt0_ref, wg0_ref, bg0_ref,
                  wt1_ref, bt1_ref, wg1_ref, bg1_ref, out_ref):
  dn = (((1,), (1,)), ((), ()))

  def mm(a, w_ref):
    return lax.dot_general(a, w_ref[...], dn,
                           preferred_element_type=jnp.float32)

  h = mm(e_ref[...], wp_ref)
  for wt_ref, bt_ref, wg_ref, bg_ref in (
      (wt0_ref, bt0_ref, wg0_ref, bg0_ref),
      (wt1_ref, bt1_ref, wg1_ref, bg1_ref)):
    zg = mm(h, wg_ref) + bg_ref[...]
    zt = mm(h, wt_ref) + bt_ref[...]
    g = 1.0 / (1.0 + jnp.exp(-zg))
    t = jnp.maximum(zt, 0.0)
    h = g * t + (1.0 - g) * h
  out_ref[...] = h


def _make_highway():
  w_spec = pl.BlockSpec((H, H), lambda i: (0, 0))
  b_spec = pl.BlockSpec((1, H), lambda i: (0, 0))
  return pl.pallas_call(
      _highway_body,
      grid=(V // TBLK,),
      in_specs=[
          pl.BlockSpec((TBLK, D), lambda i: (i, 0)),
          w_spec, w_spec, b_spec, w_spec, b_spec,
          w_spec, b_spec, w_spec, b_spec,
      ],
      out_specs=pl.BlockSpec((TBLK, H), lambda i: (i, 0)),
      out_shape=jax.ShapeDtypeStruct((V, H), jnp.float32),
  )


_highway = _make_highway()


def kernel(x, word_vectors, W_proj, Wt0, bt0, Wg0, bg0, Wt1, bt1, Wg1, bg1):
  idx = x.reshape(NW, NCH, CH).astype(jnp.int32)
  ftable = _highway(word_vectors, W_proj,
                    Wt0, bt0.reshape(1, H), Wg0, bg0.reshape(1, H),
                    Wt1, bt1.reshape(1, H), Wg1, bg1.reshape(1, H))
  out = _gather(ftable, idx)
  return out.reshape(B, L, H)
